# Initial kernel scaffold; baseline (speedup 1.0000x reference)
import jax
import jax.numpy as jnp
from jax.experimental import pallas as pl


def kernel(outputs, y):
    raise NotImplementedError("write your pallas kernel here")


# speedup vs baseline: 5406.3046x; 5406.3046x over previous
import jax
import jax.numpy as jnp
from jax.experimental import pallas as pl
from jax.experimental.pallas import tpu as pltpu

_N = 2048
_MARGIN = 0.1
_BLK = 256


def _body(orow_ref, yrow_ref, ocol_ref, ycol_ref, out_ref, acc_ref):
    i = pl.program_id(0)

    @pl.when(i == 0)
    def _init():
        acc_ref[0] = 0.0
        acc_ref[1] = 0.0

    orow = orow_ref[...]
    yrow = yrow_ref[...]
    oc = ocol_ref[...]
    yc = ycol_ref[...]
    dy = yrow - yc
    do = orow - oc
    t = jnp.sign(dy)
    per = jnp.maximum(0.0, _MARGIN - t * do)
    valid = dy != 0.0
    acc_ref[0] += jnp.sum(jnp.where(valid, per, 0.0))
    acc_ref[1] += jnp.sum(valid.astype(jnp.float32))

    @pl.when(i == pl.num_programs(0) - 1)
    def _fin():
        out_ref[...] = jnp.full((1, 1), acc_ref[0] / jnp.maximum(acc_ref[1], 1.0),
                                dtype=jnp.float32)


def kernel(outputs, y):
    o2 = outputs.reshape(_N, 1)
    y2 = y.reshape(_N, 1)
    oc = outputs.reshape(1, _N)
    yc = y.reshape(1, _N)
    grid = (_N // _BLK,)
    res = pl.pallas_call(
        _body,
        grid=grid,
        in_specs=[
            pl.BlockSpec((_BLK, 1), lambda i: (i, 0)),
            pl.BlockSpec((_BLK, 1), lambda i: (i, 0)),
            pl.BlockSpec((1, _N), lambda i: (0, 0)),
            pl.BlockSpec((1, _N), lambda i: (0, 0)),
        ],
        out_specs=pl.BlockSpec((1, 1), lambda i: (0, 0)),
        out_shape=jax.ShapeDtypeStruct((1, 1), jnp.float32),
        scratch_shapes=[pltpu.SMEM((2,), jnp.float32)],
    )(o2, y2, oc, yc)
    return res.reshape(())
